# baseline (speedup 1.0000x reference)
import functools

import jax
import jax.numpy as jnp
from jax import lax
from jax.experimental import pallas as pl
from jax.experimental.pallas import tpu as pltpu
from jax.experimental.pallas import tpu_sc as plsc

TOP_K = 100
LANE = 128
SUB = 16
ROWS = 8
KEEP_C = 128
KEEP_S = 128
NCORES = 2
NSUBCORES = 16
NW = NCORES * NSUBCORES


def _chunk_stage_kernel(x_ref, xc_ref, cid_ref, gflat_ref, cm_sc, *, nchunks, batch, block_rows):
    BIG = jnp.int32(2**30)
    i = pl.program_id(0)
    nblocks = pl.num_programs(0)
    x = x_ref[...]
    xm = x[:, :nchunks * LANE].reshape(block_rows, nchunks, LANE)
    xc_ref[...] = xm.reshape(block_rows * nchunks, LANE)
    cm_sc[pl.ds(i * block_rows, block_rows), :] = jnp.max(xm, axis=2)

    @pl.when(i == nblocks - 1)
    def _():
        cm = cm_sc[...]
        pos = lax.broadcasted_iota(jnp.int32, (batch, nchunks), 1)
        liota = lax.broadcasted_iota(jnp.int32, (batch, LANE), 1)

        def step(j, carry):
            cm, acc = carry
            m = jnp.max(cm, axis=1, keepdims=True)
            key = jnp.where(cm == m, pos, BIG)
            p = jnp.min(key, axis=1, keepdims=True)
            acc = jnp.where(liota == j, p, acc)
            cm = jnp.where(key == p, -jnp.inf, cm)
            return cm, acc

        _, cids = lax.fori_loop(
            0, KEEP_C, step, (cm, jnp.zeros((batch, LANE), jnp.int32)))
        cid_ref[...] = cids
        row = lax.broadcasted_iota(jnp.int32, (batch, LANE), 0)
        gflat_ref[...] = cids + row * nchunks


def _sub_stage_kernel(cube_ref, cid_ref, ids_ref, gy_ref, sm_sc, *, nsub, batch, block_rows):
    BIG = jnp.int32(2**30)
    i = pl.program_id(0)
    nblocks = pl.num_programs(0)
    cube = cube_ref[...]
    sm_sc[pl.ds(i * block_rows, block_rows), :] = jnp.max(
        cube.reshape(block_rows, KEEP_C, 8, SUB), axis=3).reshape(block_rows, KEEP_C * 8)

    @pl.when(i == nblocks - 1)
    def _():
        sm = sm_sc[...]
        cids = cid_ref[...]
        cid8 = jnp.broadcast_to(cids[:, :, None], (batch, KEEP_C, 8))
        sub_i = lax.broadcasted_iota(jnp.int32, (batch, KEEP_C, 8), 2)
        fullmap = (cid8 * 8 + sub_i).reshape(batch, KEEP_C * 8)
        pos = lax.broadcasted_iota(jnp.int32, (batch, KEEP_C * 8), 1)
        liota = lax.broadcasted_iota(jnp.int32, (batch, LANE), 1)

        def step(j, carry):
            sm, acc = carry
            m = jnp.max(sm, axis=1, keepdims=True)
            key = jnp.where(sm == m, pos, BIG)
            p = jnp.min(key, axis=1, keepdims=True)
            sel = key == p
            fs = jnp.min(jnp.where(sel, fullmap, BIG), axis=1, keepdims=True)
            acc = jnp.where(liota == j, fs, acc)
            sm = jnp.where(sel, -jnp.inf, sm)
            return sm, acc

        _, ids = lax.fori_loop(
            0, KEEP_S, step, (sm, jnp.zeros((batch, LANE), jnp.int32)))
        ids_ref[...] = ids
        row = lax.broadcasted_iota(jnp.int32, (batch, LANE), 0)
        gy_ref[...] = ids + row * nsub


def _final_kernel(candy_ref, candw_ref, ids_ref, ytail_ref, wtail_ref,
                  words_ref, scores_ref, *, k, batch, tail_base):
    BIG = jnp.int32(2**30)
    ngrp = KEEP_S // 8
    x = jnp.concatenate([candy_ref[...], ytail_ref[...][:, None, :]], axis=1)
    cw = jnp.concatenate([candw_ref[...], wtail_ref[...][:, None, :]], axis=1)
    ids = ids_ref[...]
    l16 = lax.broadcasted_iota(jnp.int32, (batch, ngrp, 8, SUB), 3)
    ids4 = jnp.broadcast_to(ids.reshape(batch, ngrp, 8)[:, :, :, None],
                            (batch, ngrp, 8, SUB))
    om_main = (ids4 * SUB + l16).reshape(batch, ngrp, LANE)
    om_tail = (lax.broadcasted_iota(jnp.int32, (batch, 1, LANE), 2) + tail_base)
    origmap = jnp.concatenate([om_main, om_tail], axis=1)
    liota = lax.broadcasted_iota(jnp.int32, (batch, LANE), 1)

    def step(j, carry):
        x, wacc, sacc = carry
        m = jnp.max(x, axis=(1, 2), keepdims=True)
        key = jnp.where(x == m, origmap, BIG)
        om = jnp.min(key, axis=(1, 2), keepdims=True)
        sel = key == om
        w = jnp.min(jnp.where(sel, cw, BIG), axis=(1, 2), keepdims=True)
        wacc = jnp.where(liota == j, w[:, :, 0], wacc)
        sacc = jnp.where(liota == j, m[:, :, 0], sacc)
        x = jnp.where(sel, -jnp.inf, x)
        return x, wacc, sacc

    init = (x, jnp.zeros((batch, LANE), jnp.int32),
            jnp.zeros((batch, LANE), jnp.float32))
    _, wacc, sacc = lax.fori_loop(0, k, step, init)
    words_ref[...] = wacc
    scores_ref[...] = sacc


def _sc_gather_rows(table, idx2):
    nj, _ = idx2.shape
    n = nj * LANE
    d = table.shape[1]
    bpw = n // NW
    jw = bpw // LANE
    mesh = plsc.VectorSubcoreMesh(core_axis_name="c", subcore_axis_name="s")

    @functools.partial(
        pl.kernel, mesh=mesh,
        out_type=jax.ShapeDtypeStruct((n, d), table.dtype),
        compiler_params=pltpu.CompilerParams(use_tc_tiling_on_sc=False),
        scratch_types=[
            pltpu.VMEM((jw, LANE), jnp.int32),
            pltpu.VMEM((bpw, d), table.dtype),
            pltpu.SemaphoreType.DMA,
        ],
    )
    def k(table_hbm, idx_hbm, out_hbm, idx_v, rows_v, sem):
        wid = lax.axis_index("s") * NCORES + lax.axis_index("c")
        pltpu.sync_copy(idx_hbm.at[pl.ds(wid * jw, jw)], idx_v)
        copies = [
            pltpu.make_async_copy(
                table_hbm.at[idx_v.at[j]], rows_v.at[pl.ds(j * LANE, LANE)], sem)
            for j in range(jw)
        ]
        for c in copies:
            c.start()
        for c in copies:
            c.wait()
        pltpu.sync_copy(rows_v, out_hbm.at[pl.ds(wid * bpw, bpw)])

    return k(table, idx2)


def _sc_gather2(tabley, idxy2, tablew, idxw2):
    nj, _ = idxy2.shape
    n = nj * LANE
    dy = tabley.shape[1]
    dw = tablew.shape[1]
    bpw = n // NW
    jw = bpw // LANE
    mesh = plsc.VectorSubcoreMesh(core_axis_name="c", subcore_axis_name="s")

    @functools.partial(
        pl.kernel, mesh=mesh,
        out_type=[
            jax.ShapeDtypeStruct((n, dy), tabley.dtype),
            jax.ShapeDtypeStruct((n, dw), tablew.dtype),
        ],
        compiler_params=pltpu.CompilerParams(use_tc_tiling_on_sc=False),
        scratch_types=[
            pltpu.VMEM((jw, LANE), jnp.int32),
            pltpu.VMEM((jw, LANE), jnp.int32),
            pltpu.VMEM((bpw, dy), tabley.dtype),
            pltpu.VMEM((bpw, dw), tablew.dtype),
            pltpu.SemaphoreType.DMA,
        ],
    )
    def k(ty_hbm, iy_hbm, tw_hbm, iw_hbm, oy_hbm, ow_hbm,
          iy_v, iw_v, ry_v, rw_v, sem):
        wid = lax.axis_index("s") * NCORES + lax.axis_index("c")
        pltpu.sync_copy(iy_hbm.at[pl.ds(wid * jw, jw)], iy_v)
        pltpu.sync_copy(iw_hbm.at[pl.ds(wid * jw, jw)], iw_v)
        copies = [
            pltpu.make_async_copy(
                ty_hbm.at[iy_v.at[j]], ry_v.at[pl.ds(j * LANE, LANE)], sem)
            for j in range(jw)
        ] + [
            pltpu.make_async_copy(
                tw_hbm.at[iw_v.at[j]], rw_v.at[pl.ds(j * LANE, LANE)], sem)
            for j in range(jw)
        ]
        for c in copies:
            c.start()
        for c in copies:
            c.wait()
        pltpu.sync_copy(ry_v, oy_hbm.at[pl.ds(wid * bpw, bpw)])
        pltpu.sync_copy(rw_v, ow_hbm.at[pl.ds(wid * bpw, bpw)])

    return k(tabley, idxy2, tablew, idxw2)


def kernel(y_pred, word_table):
    batch, vocab = y_pred.shape
    nchunks = vocab // LANE
    main = nchunks * LANE
    tail_n = vocab - main
    nsub = vocab // SUB

    nblocks = batch // ROWS
    chunk_table, cids, gflat = pl.pallas_call(
        functools.partial(_chunk_stage_kernel, nchunks=nchunks, batch=batch,
                          block_rows=ROWS),
        grid=(nblocks,),
        in_specs=[pl.BlockSpec((ROWS, vocab), lambda i: (i, 0))],
        out_specs=[
            pl.BlockSpec((ROWS * nchunks, LANE), lambda i: (i, 0)),
            pl.BlockSpec((batch, LANE), lambda i: (0, 0)),
            pl.BlockSpec((batch, LANE), lambda i: (0, 0)),
        ],
        out_shape=[
            jax.ShapeDtypeStruct((batch * nchunks, LANE), jnp.float32),
            jax.ShapeDtypeStruct((batch, LANE), jnp.int32),
            jax.ShapeDtypeStruct((batch, LANE), jnp.int32),
        ],
        scratch_shapes=[pltpu.VMEM((batch, nchunks), jnp.float32)],
    )(y_pred)

    cube = _sc_gather_rows(chunk_table, gflat).reshape(batch, KEEP_C, LANE)

    ids_sub, gy = pl.pallas_call(
        functools.partial(_sub_stage_kernel, nsub=nsub, batch=batch,
                          block_rows=ROWS),
        grid=(nblocks,),
        in_specs=[
            pl.BlockSpec((ROWS, KEEP_C, LANE), lambda i: (i, 0, 0)),
            pl.BlockSpec((batch, LANE), lambda i: (0, 0)),
        ],
        out_specs=[
            pl.BlockSpec((batch, LANE), lambda i: (0, 0)),
            pl.BlockSpec((batch, LANE), lambda i: (0, 0)),
        ],
        out_shape=[
            jax.ShapeDtypeStruct((batch, LANE), jnp.int32),
            jax.ShapeDtypeStruct((batch, LANE), jnp.int32),
        ],
        scratch_shapes=[pltpu.VMEM((batch, KEEP_C * 8), jnp.float32)],
    )(cube, cids)

    y_sub = y_pred.reshape(batch * nsub, SUB)
    wt_sub = word_table.reshape(nsub, SUB)
    candy, candw = _sc_gather2(y_sub, gy, wt_sub, ids_sub)
    ngrp = KEEP_S // 8
    candy = candy.reshape(batch, ngrp, LANE)
    candw = candw.reshape(batch, ngrp, LANE)

    ytail = jnp.pad(y_pred[:, main:], ((0, 0), (0, LANE - tail_n)),
                    constant_values=-jnp.inf)
    wtail = jnp.broadcast_to(
        jnp.pad(word_table[main:], (0, LANE - tail_n))[None, :], (batch, LANE))

    words, scores = pl.pallas_call(
        functools.partial(_final_kernel, k=TOP_K, batch=batch, tail_base=main),
        out_shape=[
            jax.ShapeDtypeStruct((batch, LANE), jnp.int32),
            jax.ShapeDtypeStruct((batch, LANE), jnp.float32),
        ],
    )(candy, candw, ids_sub, ytail, wtail)

    return words[:, :TOP_K], scores[:, :TOP_K]


# speedup vs baseline: 1.1837x; 1.1837x over previous
import functools

import jax
import jax.numpy as jnp
from jax import lax
from jax.experimental import pallas as pl
from jax.experimental.pallas import tpu as pltpu
from jax.experimental.pallas import tpu_sc as plsc

TOP_K = 100
LANE = 128
SUB = 16
ROWS = 16
KEEP_C = 128
KEEP_S = 128
NCORES = 2
NSUBCORES = 16
NW = NCORES * NSUBCORES


def _chunk_stage_kernel(x_ref, xc_ref, cid_ref, gflat_ref, cm_sc, *, nchunks, batch, block_rows):
    BIG = jnp.int32(2**30)
    i = pl.program_id(0)
    nblocks = pl.num_programs(0)
    x = x_ref[...]
    xm = x[:, :nchunks * LANE].reshape(block_rows, nchunks, LANE)
    xc_ref[...] = xm.reshape(block_rows * nchunks, LANE)
    cm_sc[pl.ds(i * block_rows, block_rows), :] = jnp.max(xm, axis=2)

    @pl.when(i == nblocks - 1)
    def _():
        cm = cm_sc[...]
        pos = lax.broadcasted_iota(jnp.int32, (batch, nchunks), 1)
        liota = lax.broadcasted_iota(jnp.int32, (batch, LANE), 1)

        def step(j, carry):
            cm, acc = carry
            m = jnp.max(cm, axis=1, keepdims=True)
            key = jnp.where(cm == m, pos, BIG)
            p = jnp.min(key, axis=1, keepdims=True)
            acc = jnp.where(liota == j, p, acc)
            cm = jnp.where(key == p, -jnp.inf, cm)
            return cm, acc

        _, cids = lax.fori_loop(
            0, KEEP_C, step, (cm, jnp.zeros((batch, LANE), jnp.int32)))
        cid_ref[...] = cids
        row = lax.broadcasted_iota(jnp.int32, (batch, LANE), 0)
        gflat_ref[...] = cids + row * nchunks


def _sub_stage_kernel(cube_ref, cid_ref, ids_ref, gy_ref, sm_sc, *, nsub, batch, block_rows):
    BIG = jnp.int32(2**30)
    i = pl.program_id(0)
    nblocks = pl.num_programs(0)
    cube = cube_ref[...]
    sm_sc[pl.ds(i * block_rows, block_rows), :] = jnp.max(
        cube.reshape(block_rows, KEEP_C, 8, SUB), axis=3).reshape(block_rows, KEEP_C * 8)

    @pl.when(i == nblocks - 1)
    def _():
        sm = sm_sc[...]
        cids = cid_ref[...]
        cid8 = jnp.broadcast_to(cids[:, :, None], (batch, KEEP_C, 8))
        sub_i = lax.broadcasted_iota(jnp.int32, (batch, KEEP_C, 8), 2)
        fullmap = (cid8 * 8 + sub_i).reshape(batch, KEEP_C * 8)
        pos = lax.broadcasted_iota(jnp.int32, (batch, KEEP_C * 8), 1)
        liota = lax.broadcasted_iota(jnp.int32, (batch, LANE), 1)

        def step(j, carry):
            sm, acc = carry
            m = jnp.max(sm, axis=1, keepdims=True)
            key = jnp.where(sm == m, pos, BIG)
            p = jnp.min(key, axis=1, keepdims=True)
            sel = key == p
            fs = jnp.min(jnp.where(sel, fullmap, BIG), axis=1, keepdims=True)
            acc = jnp.where(liota == j, fs, acc)
            sm = jnp.where(sel, -jnp.inf, sm)
            return sm, acc

        _, ids = lax.fori_loop(
            0, KEEP_S, step, (sm, jnp.zeros((batch, LANE), jnp.int32)))
        ids_ref[...] = ids
        row = lax.broadcasted_iota(jnp.int32, (batch, LANE), 0)
        gy_ref[...] = ids + row * nsub


def _final_kernel(candy_ref, candw_ref, ids_ref, ytail_ref, wtail_ref,
                  words_ref, scores_ref, *, k, batch, tail_base, tail_n):
    BIG = jnp.int32(2**30)
    x = jnp.concatenate([candy_ref[...], ytail_ref[...]], axis=1)
    cw = jnp.concatenate([candw_ref[...], wtail_ref[...]], axis=1)
    ids = ids_ref[...]
    l16 = lax.broadcasted_iota(jnp.int32, (batch, KEEP_S, SUB), 2)
    om_main = (jnp.broadcast_to(ids[:, :, None], (batch, KEEP_S, SUB)) * SUB
               + l16).reshape(batch, KEEP_S * SUB)
    om_tail = (lax.broadcasted_iota(jnp.int32, (batch, tail_n), 1) + tail_base)
    origmap = jnp.concatenate([om_main, om_tail], axis=1)
    liota = lax.broadcasted_iota(jnp.int32, (batch, LANE), 1)

    def step(j, carry):
        x, wacc, sacc = carry
        m = jnp.max(x, axis=1, keepdims=True)
        key = jnp.where(x == m, origmap, BIG)
        om = jnp.min(key, axis=1, keepdims=True)
        sel = key == om
        w = jnp.min(jnp.where(sel, cw, BIG), axis=1, keepdims=True)
        wacc = jnp.where(liota == j, w, wacc)
        sacc = jnp.where(liota == j, m, sacc)
        x = jnp.where(sel, -jnp.inf, x)
        return x, wacc, sacc

    init = (x, jnp.zeros((batch, LANE), jnp.int32),
            jnp.zeros((batch, LANE), jnp.float32))
    _, wacc, sacc = lax.fori_loop(0, k, step, init)
    words_ref[...] = wacc
    scores_ref[...] = sacc


def _sc_gather_rows(table, idx2):
    nj, _ = idx2.shape
    n = nj * LANE
    d = table.shape[1]
    bpw = n // NW
    jw = bpw // LANE
    mesh = plsc.VectorSubcoreMesh(core_axis_name="c", subcore_axis_name="s")

    @functools.partial(
        pl.kernel, mesh=mesh,
        out_type=jax.ShapeDtypeStruct((n, d), table.dtype),
        compiler_params=pltpu.CompilerParams(use_tc_tiling_on_sc=False),
        scratch_types=[
            pltpu.VMEM((jw, LANE), jnp.int32),
            pltpu.VMEM((bpw, d), table.dtype),
            pltpu.SemaphoreType.DMA,
        ],
    )
    def k(table_hbm, idx_hbm, out_hbm, idx_v, rows_v, sem):
        wid = lax.axis_index("s") * NCORES + lax.axis_index("c")
        pltpu.sync_copy(idx_hbm.at[pl.ds(wid * jw, jw)], idx_v)
        copies = [
            pltpu.make_async_copy(
                table_hbm.at[idx_v.at[j]], rows_v.at[pl.ds(j * LANE, LANE)], sem)
            for j in range(jw)
        ]
        for c in copies:
            c.start()
        for c in copies:
            c.wait()
        pltpu.sync_copy(rows_v, out_hbm.at[pl.ds(wid * bpw, bpw)])

    return k(table, idx2)


def _sc_gather2(tabley, idxy2, tablew, idxw2):
    nj, _ = idxy2.shape
    n = nj * LANE
    dy = tabley.shape[1]
    dw = tablew.shape[1]
    bpw = n // NW
    jw = bpw // LANE
    mesh = plsc.VectorSubcoreMesh(core_axis_name="c", subcore_axis_name="s")

    @functools.partial(
        pl.kernel, mesh=mesh,
        out_type=[
            jax.ShapeDtypeStruct((n, dy), tabley.dtype),
            jax.ShapeDtypeStruct((n, dw), tablew.dtype),
        ],
        compiler_params=pltpu.CompilerParams(use_tc_tiling_on_sc=False),
        scratch_types=[
            pltpu.VMEM((jw, LANE), jnp.int32),
            pltpu.VMEM((jw, LANE), jnp.int32),
            pltpu.VMEM((bpw, dy), tabley.dtype),
            pltpu.VMEM((bpw, dw), tablew.dtype),
            pltpu.SemaphoreType.DMA,
        ],
    )
    def k(ty_hbm, iy_hbm, tw_hbm, iw_hbm, oy_hbm, ow_hbm,
          iy_v, iw_v, ry_v, rw_v, sem):
        wid = lax.axis_index("s") * NCORES + lax.axis_index("c")
        pltpu.sync_copy(iy_hbm.at[pl.ds(wid * jw, jw)], iy_v)
        pltpu.sync_copy(iw_hbm.at[pl.ds(wid * jw, jw)], iw_v)
        copies = [
            pltpu.make_async_copy(
                ty_hbm.at[iy_v.at[j]], ry_v.at[pl.ds(j * LANE, LANE)], sem)
            for j in range(jw)
        ] + [
            pltpu.make_async_copy(
                tw_hbm.at[iw_v.at[j]], rw_v.at[pl.ds(j * LANE, LANE)], sem)
            for j in range(jw)
        ]
        for c in copies:
            c.start()
        for c in copies:
            c.wait()
        pltpu.sync_copy(ry_v, oy_hbm.at[pl.ds(wid * bpw, bpw)])
        pltpu.sync_copy(rw_v, ow_hbm.at[pl.ds(wid * bpw, bpw)])

    return k(tabley, idxy2, tablew, idxw2)


def kernel(y_pred, word_table):
    batch, vocab = y_pred.shape
    nchunks = vocab // LANE
    main = nchunks * LANE
    tail_n = vocab - main
    nsub = vocab // SUB

    nblocks = batch // ROWS
    chunk_table, cids, gflat = pl.pallas_call(
        functools.partial(_chunk_stage_kernel, nchunks=nchunks, batch=batch,
                          block_rows=ROWS),
        grid=(nblocks,),
        in_specs=[pl.BlockSpec((ROWS, vocab), lambda i: (i, 0))],
        out_specs=[
            pl.BlockSpec((ROWS * nchunks, LANE), lambda i: (i, 0)),
            pl.BlockSpec((batch, LANE), lambda i: (0, 0)),
            pl.BlockSpec((batch, LANE), lambda i: (0, 0)),
        ],
        out_shape=[
            jax.ShapeDtypeStruct((batch * nchunks, LANE), jnp.float32),
            jax.ShapeDtypeStruct((batch, LANE), jnp.int32),
            jax.ShapeDtypeStruct((batch, LANE), jnp.int32),
        ],
        scratch_shapes=[pltpu.VMEM((batch, nchunks), jnp.float32)],
    )(y_pred)

    cube = _sc_gather_rows(chunk_table, gflat).reshape(batch, KEEP_C, LANE)

    ids_sub, gy = pl.pallas_call(
        functools.partial(_sub_stage_kernel, nsub=nsub, batch=batch,
                          block_rows=ROWS),
        grid=(nblocks,),
        in_specs=[
            pl.BlockSpec((ROWS, KEEP_C, LANE), lambda i: (i, 0, 0)),
            pl.BlockSpec((batch, LANE), lambda i: (0, 0)),
        ],
        out_specs=[
            pl.BlockSpec((batch, LANE), lambda i: (0, 0)),
            pl.BlockSpec((batch, LANE), lambda i: (0, 0)),
        ],
        out_shape=[
            jax.ShapeDtypeStruct((batch, LANE), jnp.int32),
            jax.ShapeDtypeStruct((batch, LANE), jnp.int32),
        ],
        scratch_shapes=[pltpu.VMEM((batch, KEEP_C * 8), jnp.float32)],
    )(cube, cids)

    y_sub = y_pred.reshape(batch * nsub, SUB)
    wt_sub = word_table.reshape(nsub, SUB)
    candy, candw = _sc_gather2(y_sub, gy, wt_sub, ids_sub)
    candy = candy.reshape(batch, KEEP_S * SUB)
    candw = candw.reshape(batch, KEEP_S * SUB)

    ytail = y_pred[:, main:]
    wtail = jnp.broadcast_to(word_table[None, main:], (batch, tail_n))

    words, scores = pl.pallas_call(
        functools.partial(_final_kernel, k=TOP_K, batch=batch,
                          tail_base=main, tail_n=tail_n),
        out_shape=[
            jax.ShapeDtypeStruct((batch, LANE), jnp.int32),
            jax.ShapeDtypeStruct((batch, LANE), jnp.float32),
        ],
    )(candy, candw, ids_sub, ytail, wtail)

    return words[:, :TOP_K], scores[:, :TOP_K]


# speedup vs baseline: 1.3895x; 1.1738x over previous
import functools

import jax
import jax.numpy as jnp
from jax import lax
from jax.experimental import pallas as pl
from jax.experimental.pallas import tpu as pltpu
from jax.experimental.pallas import tpu_sc as plsc

TOP_K = 100
LANE = 128
SUB = 16
ROWS = 16
KEEP_C = 128
KEEP_S = 128
NCORES = 2
NSUBCORES = 16
NW = NCORES * NSUBCORES


def _chunk_stage_kernel(x_ref, xc_ref, cid_ref, gflat_ref, cm_sc, *, nchunks, batch, block_rows):
    BIG = jnp.int32(2**30)
    i = pl.program_id(0)
    nblocks = pl.num_programs(0)
    x = x_ref[...]
    xm = x[:, :nchunks * LANE].reshape(block_rows, nchunks, LANE)
    xc_ref[...] = xm.reshape(block_rows * nchunks, LANE)
    cm_sc[pl.ds(i * block_rows, block_rows), :] = jnp.max(xm, axis=2)

    @pl.when(i == nblocks - 1)
    def _():
        cm = cm_sc[...]
        pos = lax.broadcasted_iota(jnp.int32, (batch, nchunks), 1)
        liota = lax.broadcasted_iota(jnp.int32, (batch, LANE), 1)

        def step(j, carry):
            cm, acc = carry
            m = jnp.max(cm, axis=1, keepdims=True)
            key = jnp.where(cm == m, pos, BIG)
            p = jnp.min(key, axis=1, keepdims=True)
            acc = jnp.where(liota == j, p, acc)
            cm = jnp.where(key == p, -jnp.inf, cm)
            return cm, acc

        _, cids = lax.fori_loop(
            0, KEEP_C, step, (cm, jnp.zeros((batch, LANE), jnp.int32)),
            unroll=8)
        cid_ref[...] = cids
        row = lax.broadcasted_iota(jnp.int32, (batch, LANE), 0)
        gflat_ref[...] = cids + row * nchunks


def _sub_stage_kernel(cube_ref, cid_ref, ids_ref, gy_ref, sm_sc, *, nsub, batch, block_rows):
    BIG = jnp.int32(2**30)
    i = pl.program_id(0)
    nblocks = pl.num_programs(0)
    cube = cube_ref[...]
    sm_sc[pl.ds(i * block_rows, block_rows), :] = jnp.max(
        cube.reshape(block_rows, KEEP_C, 8, SUB), axis=3).reshape(block_rows, KEEP_C * 8)

    @pl.when(i == nblocks - 1)
    def _():
        sm = sm_sc[...]
        cids = cid_ref[...]
        cid8 = jnp.broadcast_to(cids[:, :, None], (batch, KEEP_C, 8))
        sub_i = lax.broadcasted_iota(jnp.int32, (batch, KEEP_C, 8), 2)
        fullmap = (cid8 * 8 + sub_i).reshape(batch, KEEP_C * 8)
        pos = lax.broadcasted_iota(jnp.int32, (batch, KEEP_C * 8), 1)
        liota = lax.broadcasted_iota(jnp.int32, (batch, LANE), 1)

        def step(j, carry):
            sm, acc = carry
            m = jnp.max(sm, axis=1, keepdims=True)
            key = jnp.where(sm == m, pos, BIG)
            p = jnp.min(key, axis=1, keepdims=True)
            sel = key == p
            fs = jnp.min(jnp.where(sel, fullmap, BIG), axis=1, keepdims=True)
            acc = jnp.where(liota == j, fs, acc)
            sm = jnp.where(sel, -jnp.inf, sm)
            return sm, acc

        _, ids = lax.fori_loop(
            0, KEEP_S, step, (sm, jnp.zeros((batch, LANE), jnp.int32)),
            unroll=8)
        ids_ref[...] = ids
        row = lax.broadcasted_iota(jnp.int32, (batch, LANE), 0)
        gy_ref[...] = ids + row * nsub


def _final_kernel(candy_ref, candw_ref, ids_ref, ytail_ref, wtail_ref,
                  words_ref, scores_ref, *, k, batch, tail_base, tail_n):
    BIG = jnp.int32(2**30)
    x = jnp.concatenate([candy_ref[...], ytail_ref[...]], axis=1)
    cw = jnp.concatenate([candw_ref[...], wtail_ref[...]], axis=1)
    ids = ids_ref[...]
    l16 = lax.broadcasted_iota(jnp.int32, (batch, KEEP_S, SUB), 2)
    om_main = (jnp.broadcast_to(ids[:, :, None], (batch, KEEP_S, SUB)) * SUB
               + l16).reshape(batch, KEEP_S * SUB)
    om_tail = (lax.broadcasted_iota(jnp.int32, (batch, tail_n), 1) + tail_base)
    origmap = jnp.concatenate([om_main, om_tail], axis=1)
    liota = lax.broadcasted_iota(jnp.int32, (batch, LANE), 1)

    def step(j, carry):
        x, wacc, sacc = carry
        m = jnp.max(x, axis=1, keepdims=True)
        key = jnp.where(x == m, origmap, BIG)
        om = jnp.min(key, axis=1, keepdims=True)
        sel = key == om
        w = jnp.min(jnp.where(sel, cw, BIG), axis=1, keepdims=True)
        wacc = jnp.where(liota == j, w, wacc)
        sacc = jnp.where(liota == j, m, sacc)
        x = jnp.where(sel, -jnp.inf, x)
        return x, wacc, sacc

    init = (x, jnp.zeros((batch, LANE), jnp.int32),
            jnp.zeros((batch, LANE), jnp.float32))
    _, wacc, sacc = lax.fori_loop(0, k, step, init, unroll=4)
    words_ref[...] = wacc
    scores_ref[...] = sacc


def _sc_gather_rows(table, idx2):
    nj, _ = idx2.shape
    n = nj * LANE
    d = table.shape[1]
    bpw = n // NW
    jw = bpw // LANE
    mesh = plsc.VectorSubcoreMesh(core_axis_name="c", subcore_axis_name="s")

    @functools.partial(
        pl.kernel, mesh=mesh,
        out_type=jax.ShapeDtypeStruct((n, d), table.dtype),
        compiler_params=pltpu.CompilerParams(use_tc_tiling_on_sc=False),
        scratch_types=[
            pltpu.VMEM((jw, LANE), jnp.int32),
            pltpu.VMEM((bpw, d), table.dtype),
            pltpu.SemaphoreType.DMA,
        ],
    )
    def k(table_hbm, idx_hbm, out_hbm, idx_v, rows_v, sem):
        wid = lax.axis_index("s") * NCORES + lax.axis_index("c")
        pltpu.sync_copy(idx_hbm.at[pl.ds(wid * jw, jw)], idx_v)
        copies = [
            pltpu.make_async_copy(
                table_hbm.at[idx_v.at[j]], rows_v.at[pl.ds(j * LANE, LANE)], sem)
            for j in range(jw)
        ]
        for c in copies:
            c.start()
        for c in copies:
            c.wait()
        pltpu.sync_copy(rows_v, out_hbm.at[pl.ds(wid * bpw, bpw)])

    return k(table, idx2)


def _sc_gather2(tabley, idxy2, tablew, idxw2):
    nj, _ = idxy2.shape
    n = nj * LANE
    dy = tabley.shape[1]
    dw = tablew.shape[1]
    bpw = n // NW
    jw = bpw // LANE
    mesh = plsc.VectorSubcoreMesh(core_axis_name="c", subcore_axis_name="s")

    @functools.partial(
        pl.kernel, mesh=mesh,
        out_type=[
            jax.ShapeDtypeStruct((n, dy), tabley.dtype),
            jax.ShapeDtypeStruct((n, dw), tablew.dtype),
        ],
        compiler_params=pltpu.CompilerParams(use_tc_tiling_on_sc=False),
        scratch_types=[
            pltpu.VMEM((jw, LANE), jnp.int32),
            pltpu.VMEM((jw, LANE), jnp.int32),
            pltpu.VMEM((bpw, dy), tabley.dtype),
            pltpu.VMEM((bpw, dw), tablew.dtype),
            pltpu.SemaphoreType.DMA,
        ],
    )
    def k(ty_hbm, iy_hbm, tw_hbm, iw_hbm, oy_hbm, ow_hbm,
          iy_v, iw_v, ry_v, rw_v, sem):
        wid = lax.axis_index("s") * NCORES + lax.axis_index("c")
        pltpu.sync_copy(iy_hbm.at[pl.ds(wid * jw, jw)], iy_v)
        pltpu.sync_copy(iw_hbm.at[pl.ds(wid * jw, jw)], iw_v)
        copies = [
            pltpu.make_async_copy(
                ty_hbm.at[iy_v.at[j]], ry_v.at[pl.ds(j * LANE, LANE)], sem)
            for j in range(jw)
        ] + [
            pltpu.make_async_copy(
                tw_hbm.at[iw_v.at[j]], rw_v.at[pl.ds(j * LANE, LANE)], sem)
            for j in range(jw)
        ]
        for c in copies:
            c.start()
        for c in copies:
            c.wait()
        pltpu.sync_copy(ry_v, oy_hbm.at[pl.ds(wid * bpw, bpw)])
        pltpu.sync_copy(rw_v, ow_hbm.at[pl.ds(wid * bpw, bpw)])

    return k(tabley, idxy2, tablew, idxw2)


def kernel(y_pred, word_table):
    batch, vocab = y_pred.shape
    nchunks = vocab // LANE
    main = nchunks * LANE
    tail_n = vocab - main
    nsub = vocab // SUB

    nblocks = batch // ROWS
    chunk_table, cids, gflat = pl.pallas_call(
        functools.partial(_chunk_stage_kernel, nchunks=nchunks, batch=batch,
                          block_rows=ROWS),
        grid=(nblocks,),
        in_specs=[pl.BlockSpec((ROWS, vocab), lambda i: (i, 0))],
        out_specs=[
            pl.BlockSpec((ROWS * nchunks, LANE), lambda i: (i, 0)),
            pl.BlockSpec((batch, LANE), lambda i: (0, 0)),
            pl.BlockSpec((batch, LANE), lambda i: (0, 0)),
        ],
        out_shape=[
            jax.ShapeDtypeStruct((batch * nchunks, LANE), jnp.float32),
            jax.ShapeDtypeStruct((batch, LANE), jnp.int32),
            jax.ShapeDtypeStruct((batch, LANE), jnp.int32),
        ],
        scratch_shapes=[pltpu.VMEM((batch, nchunks), jnp.float32)],
    )(y_pred)

    cube = _sc_gather_rows(chunk_table, gflat).reshape(batch, KEEP_C, LANE)

    ids_sub, gy = pl.pallas_call(
        functools.partial(_sub_stage_kernel, nsub=nsub, batch=batch,
                          block_rows=ROWS),
        grid=(nblocks,),
        in_specs=[
            pl.BlockSpec((ROWS, KEEP_C, LANE), lambda i: (i, 0, 0)),
            pl.BlockSpec((batch, LANE), lambda i: (0, 0)),
        ],
        out_specs=[
            pl.BlockSpec((batch, LANE), lambda i: (0, 0)),
            pl.BlockSpec((batch, LANE), lambda i: (0, 0)),
        ],
        out_shape=[
            jax.ShapeDtypeStruct((batch, LANE), jnp.int32),
            jax.ShapeDtypeStruct((batch, LANE), jnp.int32),
        ],
        scratch_shapes=[pltpu.VMEM((batch, KEEP_C * 8), jnp.float32)],
    )(cube, cids)

    y_sub = y_pred.reshape(batch * nsub, SUB)
    wt_sub = word_table.reshape(nsub, SUB)
    candy, candw = _sc_gather2(y_sub, gy, wt_sub, ids_sub)
    candy = candy.reshape(batch, KEEP_S * SUB)
    candw = candw.reshape(batch, KEEP_S * SUB)

    ytail = y_pred[:, main:]
    wtail = jnp.broadcast_to(word_table[None, main:], (batch, tail_n))

    words, scores = pl.pallas_call(
        functools.partial(_final_kernel, k=TOP_K, batch=batch,
                          tail_base=main, tail_n=tail_n),
        out_shape=[
            jax.ShapeDtypeStruct((batch, LANE), jnp.int32),
            jax.ShapeDtypeStruct((batch, LANE), jnp.float32),
        ],
    )(candy, candw, ids_sub, ytail, wtail)

    return words[:, :TOP_K], scores[:, :TOP_K]


# speedup vs baseline: 1.3976x; 1.0059x over previous
import functools

import jax
import jax.numpy as jnp
from jax import lax
from jax.experimental import pallas as pl
from jax.experimental.pallas import tpu as pltpu
from jax.experimental.pallas import tpu_sc as plsc

TOP_K = 100
LANE = 128
SUB = 16
ROWS = 16
KEEP_C = 128
KEEP_S = 128
NCORES = 2
NSUBCORES = 16
NW = NCORES * NSUBCORES


def _chunk_stage_kernel(x_ref, xc_ref, cid_ref, gflat_ref, cm_sc, *, nchunks, batch, block_rows):
    BIG = jnp.int32(2**30)
    i = pl.program_id(0)
    nblocks = pl.num_programs(0)
    x = x_ref[...]
    xm = x[:, :nchunks * LANE].reshape(block_rows, nchunks, LANE)
    xc_ref[...] = xm.reshape(block_rows * nchunks, LANE)
    cm_sc[pl.ds(i * block_rows, block_rows), :] = jnp.max(xm, axis=2)

    @pl.when(i == nblocks - 1)
    def _():
        cm = cm_sc[...]
        pos = lax.broadcasted_iota(jnp.int32, (batch, nchunks), 1)
        liota = lax.broadcasted_iota(jnp.int32, (batch, LANE), 1)

        def step(j, carry):
            cm, acc = carry
            m = jnp.max(cm, axis=1, keepdims=True)
            key = jnp.where(cm == m, pos, BIG)
            p = jnp.min(key, axis=1, keepdims=True)
            acc = jnp.where(liota == j, p, acc)
            cm = jnp.where(key == p, -jnp.inf, cm)
            return cm, acc

        _, cids = lax.fori_loop(
            0, KEEP_C, step, (cm, jnp.zeros((batch, LANE), jnp.int32)),
            unroll=8)
        cid_ref[...] = cids
        row = lax.broadcasted_iota(jnp.int32, (batch, LANE), 0)
        gflat_ref[...] = cids + row * nchunks


def _sub_stage_kernel(cube_ref, cid_ref, ids_ref, gy_ref, sm_sc, *, nsub, batch, block_rows):
    BIG = jnp.int32(2**30)
    i = pl.program_id(0)
    nblocks = pl.num_programs(0)
    cube = cube_ref[...]
    sm_sc[pl.ds(i * block_rows, block_rows), :] = jnp.max(
        cube.reshape(block_rows, KEEP_C, 8, SUB), axis=3).reshape(block_rows, KEEP_C * 8)

    @pl.when(i == nblocks - 1)
    def _():
        sm = sm_sc[...]
        cids = cid_ref[...]
        cid8 = jnp.broadcast_to(cids[:, :, None], (batch, KEEP_C, 8))
        sub_i = lax.broadcasted_iota(jnp.int32, (batch, KEEP_C, 8), 2)
        fullmap = (cid8 * 8 + sub_i).reshape(batch, KEEP_C * 8)
        pos = lax.broadcasted_iota(jnp.int32, (batch, KEEP_C * 8), 1)
        liota = lax.broadcasted_iota(jnp.int32, (batch, LANE), 1)

        def step(j, carry):
            sm, acc = carry
            m = jnp.max(sm, axis=1, keepdims=True)
            key = jnp.where(sm == m, pos, BIG)
            p = jnp.min(key, axis=1, keepdims=True)
            sel = key == p
            fs = jnp.min(jnp.where(sel, fullmap, BIG), axis=1, keepdims=True)
            acc = jnp.where(liota == j, fs, acc)
            sm = jnp.where(sel, -jnp.inf, sm)
            return sm, acc

        _, ids = lax.fori_loop(
            0, KEEP_S, step, (sm, jnp.zeros((batch, LANE), jnp.int32)),
            unroll=8)
        ids_ref[...] = ids
        row = lax.broadcasted_iota(jnp.int32, (batch, LANE), 0)
        gy_ref[...] = ids + row * nsub


def _final_kernel(candy_ref, candw_ref, ids_ref, ytail_ref, wtail_ref,
                  words_ref, scores_ref, *, k, batch, tail_base, tail_n):
    BIG = jnp.int32(2**30)
    x = jnp.concatenate([candy_ref[...], ytail_ref[...]], axis=1)
    cw = jnp.concatenate([candw_ref[...], wtail_ref[...]], axis=1)
    ids = ids_ref[...]
    l16 = lax.broadcasted_iota(jnp.int32, (batch, KEEP_S, SUB), 2)
    om_main = (jnp.broadcast_to(ids[:, :, None], (batch, KEEP_S, SUB)) * SUB
               + l16).reshape(batch, KEEP_S * SUB)
    om_tail = (lax.broadcasted_iota(jnp.int32, (batch, tail_n), 1) + tail_base)
    origmap = jnp.concatenate([om_main, om_tail], axis=1)
    liota = lax.broadcasted_iota(jnp.int32, (batch, LANE), 1)

    def step(j, carry):
        x, wacc, sacc = carry
        m = jnp.max(x, axis=1, keepdims=True)
        key = jnp.where(x == m, origmap, BIG)
        om = jnp.min(key, axis=1, keepdims=True)
        sel = key == om
        w = jnp.min(jnp.where(sel, cw, BIG), axis=1, keepdims=True)
        wacc = jnp.where(liota == j, w, wacc)
        sacc = jnp.where(liota == j, m, sacc)
        x = jnp.where(sel, -jnp.inf, x)
        return x, wacc, sacc

    init = (x, jnp.zeros((batch, LANE), jnp.int32),
            jnp.zeros((batch, LANE), jnp.float32))
    _, wacc, sacc = lax.fori_loop(0, k, step, init, unroll=8)
    words_ref[...] = wacc
    scores_ref[...] = sacc


def _sc_gather_rows(table, idx2):
    nj, _ = idx2.shape
    n = nj * LANE
    d = table.shape[1]
    bpw = n // NW
    jw = bpw // LANE
    mesh = plsc.VectorSubcoreMesh(core_axis_name="c", subcore_axis_name="s")

    @functools.partial(
        pl.kernel, mesh=mesh,
        out_type=jax.ShapeDtypeStruct((n, d), table.dtype),
        compiler_params=pltpu.CompilerParams(use_tc_tiling_on_sc=False),
        scratch_types=[
            pltpu.VMEM((jw, LANE), jnp.int32),
            pltpu.VMEM((bpw, d), table.dtype),
            pltpu.SemaphoreType.DMA,
        ],
    )
    def k(table_hbm, idx_hbm, out_hbm, idx_v, rows_v, sem):
        wid = lax.axis_index("s") * NCORES + lax.axis_index("c")
        pltpu.sync_copy(idx_hbm.at[pl.ds(wid * jw, jw)], idx_v)
        copies = [
            pltpu.make_async_copy(
                table_hbm.at[idx_v.at[j]], rows_v.at[pl.ds(j * LANE, LANE)], sem)
            for j in range(jw)
        ]
        for c in copies:
            c.start()
        for c in copies:
            c.wait()
        pltpu.sync_copy(rows_v, out_hbm.at[pl.ds(wid * bpw, bpw)])

    return k(table, idx2)


def _sc_gather2(tabley, idxy2, tablew, idxw2):
    nj, _ = idxy2.shape
    n = nj * LANE
    dy = tabley.shape[1]
    dw = tablew.shape[1]
    bpw = n // NW
    jw = bpw // LANE
    mesh = plsc.VectorSubcoreMesh(core_axis_name="c", subcore_axis_name="s")

    @functools.partial(
        pl.kernel, mesh=mesh,
        out_type=[
            jax.ShapeDtypeStruct((n, dy), tabley.dtype),
            jax.ShapeDtypeStruct((n, dw), tablew.dtype),
        ],
        compiler_params=pltpu.CompilerParams(use_tc_tiling_on_sc=False),
        scratch_types=[
            pltpu.VMEM((jw, LANE), jnp.int32),
            pltpu.VMEM((jw, LANE), jnp.int32),
            pltpu.VMEM((bpw, dy), tabley.dtype),
            pltpu.VMEM((bpw, dw), tablew.dtype),
            pltpu.SemaphoreType.DMA,
        ],
    )
    def k(ty_hbm, iy_hbm, tw_hbm, iw_hbm, oy_hbm, ow_hbm,
          iy_v, iw_v, ry_v, rw_v, sem):
        wid = lax.axis_index("s") * NCORES + lax.axis_index("c")
        pltpu.sync_copy(iy_hbm.at[pl.ds(wid * jw, jw)], iy_v)
        pltpu.sync_copy(iw_hbm.at[pl.ds(wid * jw, jw)], iw_v)
        copies = [
            pltpu.make_async_copy(
                ty_hbm.at[iy_v.at[j]], ry_v.at[pl.ds(j * LANE, LANE)], sem)
            for j in range(jw)
        ] + [
            pltpu.make_async_copy(
                tw_hbm.at[iw_v.at[j]], rw_v.at[pl.ds(j * LANE, LANE)], sem)
            for j in range(jw)
        ]
        for c in copies:
            c.start()
        for c in copies:
            c.wait()
        pltpu.sync_copy(ry_v, oy_hbm.at[pl.ds(wid * bpw, bpw)])
        pltpu.sync_copy(rw_v, ow_hbm.at[pl.ds(wid * bpw, bpw)])

    return k(tabley, idxy2, tablew, idxw2)


def kernel(y_pred, word_table):
    batch, vocab = y_pred.shape
    nchunks = vocab // LANE
    main = nchunks * LANE
    tail_n = vocab - main
    nsub = vocab // SUB

    nblocks = batch // ROWS
    chunk_table, cids, gflat = pl.pallas_call(
        functools.partial(_chunk_stage_kernel, nchunks=nchunks, batch=batch,
                          block_rows=ROWS),
        grid=(nblocks,),
        in_specs=[pl.BlockSpec((ROWS, vocab), lambda i: (i, 0))],
        out_specs=[
            pl.BlockSpec((ROWS * nchunks, LANE), lambda i: (i, 0)),
            pl.BlockSpec((batch, LANE), lambda i: (0, 0)),
            pl.BlockSpec((batch, LANE), lambda i: (0, 0)),
        ],
        out_shape=[
            jax.ShapeDtypeStruct((batch * nchunks, LANE), jnp.float32),
            jax.ShapeDtypeStruct((batch, LANE), jnp.int32),
            jax.ShapeDtypeStruct((batch, LANE), jnp.int32),
        ],
        scratch_shapes=[pltpu.VMEM((batch, nchunks), jnp.float32)],
    )(y_pred)

    cube = _sc_gather_rows(chunk_table, gflat).reshape(batch, KEEP_C, LANE)

    ids_sub, gy = pl.pallas_call(
        functools.partial(_sub_stage_kernel, nsub=nsub, batch=batch,
                          block_rows=ROWS),
        grid=(nblocks,),
        in_specs=[
            pl.BlockSpec((ROWS, KEEP_C, LANE), lambda i: (i, 0, 0)),
            pl.BlockSpec((batch, LANE), lambda i: (0, 0)),
        ],
        out_specs=[
            pl.BlockSpec((batch, LANE), lambda i: (0, 0)),
            pl.BlockSpec((batch, LANE), lambda i: (0, 0)),
        ],
        out_shape=[
            jax.ShapeDtypeStruct((batch, LANE), jnp.int32),
            jax.ShapeDtypeStruct((batch, LANE), jnp.int32),
        ],
        scratch_shapes=[pltpu.VMEM((batch, KEEP_C * 8), jnp.float32)],
    )(cube, cids)

    y_sub = y_pred.reshape(batch * nsub, SUB)
    wt_sub = word_table.reshape(nsub, SUB)
    candy, candw = _sc_gather2(y_sub, gy, wt_sub, ids_sub)
    candy = candy.reshape(batch, KEEP_S * SUB)
    candw = candw.reshape(batch, KEEP_S * SUB)

    ytail = y_pred[:, main:]
    wtail = jnp.broadcast_to(word_table[None, main:], (batch, tail_n))

    words, scores = pl.pallas_call(
        functools.partial(_final_kernel, k=TOP_K, batch=batch,
                          tail_base=main, tail_n=tail_n),
        out_shape=[
            jax.ShapeDtypeStruct((batch, LANE), jnp.int32),
            jax.ShapeDtypeStruct((batch, LANE), jnp.float32),
        ],
    )(candy, candw, ids_sub, ytail, wtail)

    return words[:, :TOP_K], scores[:, :TOP_K]
